# baseline (speedup 1.0000x reference)
import functools

import jax
import jax.numpy as jnp
from jax import lax
from jax.experimental import pallas as pl
from jax.experimental.pallas import tpu as pltpu
from jax.experimental.pallas import tpu_sc as plsc

RCHUNK = 160
NBUF = 3
NOUT = 2
CHUNK = 128


@functools.lru_cache(maxsize=None)
def _build_repack(V, D):
    info = plsc.get_sparse_core_info()
    NC, NS = info.num_cores, info.num_subcores
    NW = NC * NS
    assert V % RCHUNK == 0
    n_chunks = V // RCHUNK
    mesh = plsc.VectorSubcoreMesh(core_axis_name="c", subcore_axis_name="s")

    @functools.partial(
        pl.kernel,
        mesh=mesh,
        out_type=jax.ShapeDtypeStruct((V, 2 * D), jnp.float32),
        scratch_types=(
            [pltpu.VMEM((RCHUNK, D), jnp.float32) for _ in range(2)]
            + [pltpu.VMEM((RCHUNK, 2 * D), jnp.float32) for _ in range(2)]
            + [pltpu.SemaphoreType.DMA for _ in range(4)]
        ),
    )
    def repack(tok_hbm, out_hbm, in0, in1, wd0, wd1, li0, li1, ls0, ls1):
        ins = (in0, in1)
        wds = (wd0, wd1)
        lsem = (li0, li1)
        ssem = (ls0, ls1)
        wid = lax.axis_index("s") * NC + lax.axis_index("c")
        n_mine = lax.div(n_chunks - 1 - wid, NW) + 1

        def start_load(k, b):
            pltpu.make_async_copy(
                tok_hbm.at[pl.ds((wid + k * NW) * RCHUNK, RCHUNK)],
                ins[b], lsem[b]).start()

        def wait_load(b):
            pltpu.make_async_copy(
                tok_hbm.at[pl.ds(0, RCHUNK)], ins[b], lsem[b]).wait()

        def start_store(k, b):
            pltpu.make_async_copy(
                wds[b], out_hbm.at[pl.ds((wid + k * NW) * RCHUNK, RCHUNK)],
                ssem[b]).start()

        def wait_store(b):
            pltpu.make_async_copy(
                wds[b], out_hbm.at[pl.ds(0, RCHUNK)], ssem[b]).wait()

        def widen(b):
            def row(i, c):
                for q in range(D // 16):
                    sl = pl.ds(q * 16, 16)
                    wds[b][i, sl] = ins[b][i, sl]
                return c
            lax.fori_loop(0, RCHUNK, row, 0, unroll=4)

        start_load(0, 0)

        def step(k, c):
            b = lax.rem(k, 2)

            @pl.when(b == 0)
            def _():
                _one(k, 0)

            @pl.when(b == 1)
            def _():
                _one(k, 1)
            return c

        def _one(k, b):
            nb = 1 - b
            @pl.when(k + 1 < n_mine)
            def _():
                @pl.when(k >= 1)
                def _():
                    wait_store(nb)
                start_load(k + 1, nb)
            wait_load(b)
            widen(b)
            start_store(k, b)

        lax.fori_loop(0, n_mine, step, 0)
        wait_store(0)

        @pl.when(n_mine >= 2)
        def _():
            wait_store(1)

    return repack


@functools.lru_cache(maxsize=None)
def _build_sc_embed(BL, L, D):
    info = plsc.get_sparse_core_info()
    NC, NS = info.num_cores, info.num_subcores
    NW = NC * NS
    assert D == 64 and L % 2 == 0
    per_w = BL // NW
    assert BL % (NW * CHUNK) == 0 and per_w % L == 0
    n_chunks = per_w // CHUNK
    period = NBUF * NOUT
    assert (n_chunks - 2) % period == 0 and n_chunks >= period + 2
    n_packs = (n_chunks - 2) // period
    mesh = plsc.VectorSubcoreMesh(core_axis_name="c", subcore_axis_name="s")

    @functools.partial(
        pl.kernel,
        mesh=mesh,
        out_type=jax.ShapeDtypeStruct((BL, D), jnp.float32),
        scratch_types=(
            [pltpu.VMEM((n_chunks, CHUNK), jnp.int32),
             pltpu.VMEM((L // 2, 2 * D), jnp.float32)]
            + [pltpu.VMEM((CHUNK, 2 * D), jnp.float32) for _ in range(NBUF)]
            + [pltpu.VMEM((CHUNK, D), jnp.float32) for _ in range(NOUT)]
            + [pltpu.SemaphoreType.DMA for _ in range(NBUF + NOUT)]
        ),
    )
    def embed(x_hbm, tok_hbm, pos_hbm, out_hbm, idx_v, pos_v, *refs):
        gath = refs[:NBUF]
        outb = refs[NBUF:NBUF + NOUT]
        gsem = refs[NBUF + NOUT:2 * NBUF + NOUT]
        ssem = refs[2 * NBUF + NOUT:]
        wid = lax.axis_index("s") * NC + lax.axis_index("c")
        base = wid * per_w

        pltpu.sync_copy(x_hbm.at[wid], idx_v)
        pltpu.sync_copy(pos_hbm, pos_v)

        def start_gather(g, b):
            pltpu.make_async_copy(
                tok_hbm.at[idx_v.at[g]], gath[b], gsem[b]).start()

        def wait_gather(g, b):
            pltpu.make_async_copy(
                tok_hbm.at[idx_v.at[g]], gath[b], gsem[b]).wait()

        def start_store(g, o):
            pltpu.make_async_copy(
                outb[o], out_hbm.at[pl.ds(base + g * CHUNK, CHUNK)],
                ssem[o]).start()

        def wait_store(o):
            pltpu.make_async_copy(
                outb[o], out_hbm.at[pl.ds(base, CHUNK)], ssem[o]).wait()

        def compute(g, b, o):
            p0 = lax.rem(g * CHUNK, L)

            def row(i, p):
                ph = lax.shift_right_logical(p, 1)
                pc = lax.shift_left(p & 1, 6)
                for q in range(D // 16):
                    outb[o][i, pl.ds(q * 16, 16)] = (
                        gath[b][i, pl.ds(q * 16, 16)]
                        + pos_v[ph, pl.ds(pc + q * 16, 16)])
                pn = p + 1
                return jnp.where(pn == L, 0, pn)

            lax.fori_loop(0, CHUNK, row, p0, unroll=4)

        def body(g, b, o, prefetch, store_wait):
            if prefetch:
                start_gather(g + 2, (b + 2) % NBUF)
            wait_gather(g, b)
            if store_wait:
                wait_store(o)
            compute(g, b, o)
            start_store(g, o)

        start_gather(0, 0)
        start_gather(1, 1)
        body(0, 0, 0, True, False)
        body(1, 1, 1, True, False)

        def pack(pk, c):
            g0 = pk * period + 2
            for j in range(period):
                body(g0 + j, (2 + j) % NBUF, j % NOUT, True, True)
            return c

        lax.fori_loop(0, n_packs - 1, pack, 0)

        g0 = (n_packs - 1) * period + 2
        for j in range(period):
            body(g0 + j, (2 + j) % NBUF, j % NOUT, g0 + j + 2 < n_chunks, True)
        for o in range(NOUT):
            wait_store(o)

    return embed


def kernel(x, token_table, pos_table):
    B, L = x.shape
    V, D = token_table.shape
    BL = B * L
    info = plsc.get_sparse_core_info()
    NW = info.num_cores * info.num_subcores
    x_r = x.astype(jnp.int32).reshape(NW, BL // (NW * CHUNK), CHUNK)
    tok_w = _build_repack(V, D)(token_table)
    pos2 = pos_table.reshape(L // 2, 2 * D)
    out = _build_sc_embed(BL, L, D)(x_r, tok_w, pos2)
    return out.reshape(B, L, D)


# speedup vs baseline: 1.3178x; 1.3178x over previous
import functools

import jax
import jax.numpy as jnp
from jax import lax
from jax.experimental import pallas as pl
from jax.experimental.pallas import tpu as pltpu
from jax.experimental.pallas import tpu_sc as plsc

NBUF = 4


@functools.lru_cache(maxsize=None)
def _build_sc_embed(B, L, D):
    info = plsc.get_sparse_core_info()
    NC, NS = info.num_cores, info.num_subcores
    NW = NC * NS
    BL = B * L
    CHUNK = L // 2
    assert L % 2 == 0 and D % 16 == 0
    assert BL % (NW * L) == 0
    seq_w = B // NW
    per_w = BL // NW
    n_chunks = per_w // CHUNK
    assert n_chunks % NBUF == 0 and n_chunks >= 2 * NBUF
    n_quads = n_chunks // NBUF
    mesh = plsc.VectorSubcoreMesh(core_axis_name="c", subcore_axis_name="s")

    @functools.partial(
        pl.kernel,
        mesh=mesh,
        compiler_params=pltpu.CompilerParams(use_tc_tiling_on_sc=False),
        out_type=jax.ShapeDtypeStruct((B, L, D), jnp.float32),
        scratch_types=(
            [pltpu.VMEM((n_chunks, CHUNK), jnp.int32),
             pltpu.VMEM((L, D), jnp.float32)]
            + [pltpu.VMEM((CHUNK, D), jnp.float32) for _ in range(NBUF)]
            + [pltpu.SemaphoreType.DMA for _ in range(2 * NBUF)]
        ),
    )
    def embed(x_hbm, tok_hbm, pos_hbm, out_hbm, idx_v, pos_v, *bufs_and_sems):
        rows = bufs_and_sems[:NBUF]
        gsem = bufs_and_sems[NBUF:2 * NBUF]
        ssem = bufs_and_sems[2 * NBUF:]
        wid = lax.axis_index("s") * NC + lax.axis_index("c")
        seq0 = wid * seq_w

        pltpu.sync_copy(x_hbm.at[wid], idx_v)
        pltpu.sync_copy(pos_hbm, pos_v)

        def start_gather(g, b):
            pltpu.make_async_copy(
                tok_hbm.at[idx_v.at[g]], rows[b], gsem[b]).start()

        def wait_gather(g, b):
            pltpu.make_async_copy(
                tok_hbm.at[idx_v.at[g]], rows[b], gsem[b]).wait()

        def start_store(g, b, half):
            pltpu.make_async_copy(
                rows[b],
                out_hbm.at[seq0 + lax.div(g, 2), pl.ds(half * CHUNK, CHUNK)],
                ssem[b]).start()

        def wait_store(b):
            pltpu.make_async_copy(
                rows[b], out_hbm.at[seq0, pl.ds(0, CHUNK)], ssem[b]).wait()

        def add_pos(b, half):
            prow = half * CHUNK

            def row_body(i, c):
                for q in range(D // 16):
                    sl = pl.ds(q * 16, 16)
                    plsc.addupdate(rows[b].at[i, sl], pos_v[prow + i, sl])
                return c

            lax.fori_loop(0, CHUNK, row_body, 0, unroll=4)

        def chunk_body(g, b, half, prefetch, prefetch_wait):
            wait_gather(g, b)
            add_pos(b, half)
            start_store(g, b, half)
            if prefetch:
                nb = (b + NBUF - 1) % NBUF
                if prefetch_wait:
                    wait_store(nb)
                start_gather(g + NBUF - 1, nb)

        for b in range(NBUF - 1):
            start_gather(b, b)
        chunk_body(0, 0, 0, True, False)
        for b in range(1, NBUF):
            chunk_body(b, b, b % 2, True, True)

        def quad(p, c):
            g0 = p * NBUF
            for b in range(NBUF):
                chunk_body(g0 + b, b, b % 2, True, True)
            return c

        lax.fori_loop(1, n_quads - 1, quad, 0)

        g0 = (n_quads - 1) * NBUF
        chunk_body(g0, 0, 0, True, True)
        for b in range(1, NBUF):
            chunk_body(g0 + b, b, b % 2, False, False)
        for b in range(NBUF):
            wait_store(b)

    return embed


def kernel(x, token_table, pos_table):
    B, L = x.shape
    D = token_table.shape[1]
    BL = B * L
    info = plsc.get_sparse_core_info()
    NW = info.num_cores * info.num_subcores
    CHUNK = L // 2
    x_r = x.astype(jnp.int32).reshape(NW, BL // (NW * CHUNK), CHUNK)
    return _build_sc_embed(B, L, D)(x_r, token_table, pos_table)
